# baseline (speedup 1.0000x reference)
import functools

import jax
import jax.numpy as jnp
from jax import lax
from jax.experimental import pallas as pl
from jax.experimental.pallas import tpu as pltpu
from jax.experimental.pallas import tpu_sc as plsc

L = 16


@jax.jit
def kernel(x, perm):
    N, D = x.shape
    info = plsc.get_sparse_core_info()
    NC, NS = info.num_cores, info.num_subcores
    NW = NC * NS
    assert N % NW == 0
    RW = N // NW
    R_BLK = 8
    NBLK = RW // R_BLK
    NJ = D // L
    BE = R_BLK * D
    assert NBLK % 2 == 0

    mesh = plsc.VectorSubcoreMesh(core_axis_name="c", subcore_axis_name="s")

    @functools.partial(
        pl.kernel,
        out_type=jax.ShapeDtypeStruct((N * D,), jnp.float32),
        mesh=mesh,
        compiler_params=pltpu.CompilerParams(
            needs_layout_passes=False, disable_bounds_checks=True),
        scratch_types=[
            pltpu.VMEM((D,), jnp.int32),
            pltpu.VMEM((BE,), jnp.float32),
            pltpu.VMEM((BE,), jnp.float32),
            pltpu.VMEM((BE,), jnp.float32),
            pltpu.VMEM((BE,), jnp.float32),
            pltpu.SemaphoreType.DMA,
            pltpu.SemaphoreType.DMA,
            pltpu.SemaphoreType.DMA,
            pltpu.SemaphoreType.DMA,
        ],
    )
    def k(x_hbm, perm_hbm, out_hbm, perm_v, in0, in1, ou0, ou1,
          is0, is1, os0, os1):
        wid = lax.axis_index("s") * NC + lax.axis_index("c")
        base = wid * RW * D
        ins = (in0, in1)
        outs = (ou0, ou1)
        isems = (is0, is1)
        osems = (os0, os1)

        pltpu.sync_copy(perm_hbm, perm_v)

        def start_in(blk, b):
            pltpu.async_copy(
                x_hbm.at[pl.ds(base + blk * BE, BE)], ins[b], isems[b])

        def wait_in(b):
            pltpu.make_async_copy(
                x_hbm.at[pl.ds(base, BE)], ins[b], isems[b]).wait()

        def start_out(blk, b):
            pltpu.async_copy(
                outs[b], out_hbm.at[pl.ds(base + blk * BE, BE)], osems[b])

        def wait_out(b):
            pltpu.make_async_copy(
                outs[b], out_hbm.at[pl.ds(base, BE)], osems[b]).wait()

        def compute(b):
            ib = ins[b]
            ob = outs[b]

            @pl.loop(0, NJ, unroll=4)
            def j_loop(j):
                col = perm_v[pl.ds(j * L, L)]
                o = j * L
                for r in range(R_BLK):
                    v = plsc.load_gather(ib, [col + (r * D)])
                    ob[pl.ds(o + r * D, L)] = v

        start_in(0, 0)

        @pl.loop(0, NBLK, step=2)
        def blk_loop(blk0):
            for b in range(2):
                blk = blk0 + b
                nxt = jnp.minimum(blk + 1, NBLK - 1)
                start_in(nxt, 1 - b)
                wait_in(b)
                compute(b)

                @pl.when(blk >= 2)
                def _():
                    wait_out(b)

                start_out(blk, b)

        wait_in(0)
        wait_out(0)
        wait_out(1)

    return k(x.reshape(N * D), perm).reshape(N, D)


# speedup vs baseline: 1.5931x; 1.5931x over previous
import functools

import jax
import jax.numpy as jnp
from jax import lax
from jax.experimental import pallas as pl
from jax.experimental.pallas import tpu as pltpu
from jax.experimental.pallas import tpu_sc as plsc

L = 16


@jax.jit
def kernel(x, perm):
    N, D = x.shape
    info = plsc.get_sparse_core_info()
    NC, NS = info.num_cores, info.num_subcores
    NW = NC * NS
    assert N % NW == 0
    RW = N // NW
    R_BLK = 8
    NBLK = RW // R_BLK
    NJ = D // L
    assert NBLK % 2 == 0

    mesh = plsc.VectorSubcoreMesh(core_axis_name="c", subcore_axis_name="s")

    @functools.partial(
        pl.kernel,
        out_type=jax.ShapeDtypeStruct((N, D), jnp.float32),
        mesh=mesh,
        compiler_params=pltpu.CompilerParams(
            needs_layout_passes=False, disable_bounds_checks=True),
        scratch_types=[
            pltpu.VMEM((D,), jnp.int32),
            pltpu.VMEM((R_BLK, D), jnp.float32),
            pltpu.VMEM((R_BLK, D), jnp.float32),
            pltpu.VMEM((R_BLK, D), jnp.float32),
            pltpu.VMEM((R_BLK, D), jnp.float32),
            pltpu.SemaphoreType.DMA,
            pltpu.SemaphoreType.DMA,
            pltpu.SemaphoreType.DMA,
            pltpu.SemaphoreType.DMA,
        ],
    )
    def k(x_hbm, perm_hbm, out_hbm, perm_v, in0, in1, ou0, ou1,
          is0, is1, os0, os1):
        wid = lax.axis_index("s") * NC + lax.axis_index("c")
        base = wid * RW
        ins = (in0, in1)
        outs = (ou0, ou1)
        isems = (is0, is1)
        osems = (os0, os1)

        pltpu.sync_copy(perm_hbm, perm_v)

        def start_in(blk, b):
            pltpu.async_copy(
                x_hbm.at[pl.ds(base + blk * R_BLK, R_BLK)], ins[b], isems[b])

        def wait_in(b):
            pltpu.make_async_copy(
                x_hbm.at[pl.ds(base, R_BLK)], ins[b], isems[b]).wait()

        def start_out(blk, b):
            pltpu.async_copy(
                outs[b], out_hbm.at[pl.ds(base + blk * R_BLK, R_BLK)],
                osems[b])

        def wait_out(b):
            pltpu.make_async_copy(
                outs[b], out_hbm.at[pl.ds(base, R_BLK)], osems[b]).wait()

        def compute(b):
            ib = ins[b]
            ob = outs[b]

            @pl.loop(0, NJ, unroll=4)
            def j_loop(j):
                col = perm_v[pl.ds(j * L, L)]
                for r in range(R_BLK):
                    rowv = jnp.full((L,), r, jnp.int32)
                    v = plsc.load_gather(ib, [rowv, col])
                    ob[r, pl.ds(j * L, L)] = v

        start_in(0, 0)

        @pl.loop(0, NBLK, step=2)
        def blk_loop(blk0):
            for b in range(2):
                blk = blk0 + b
                nxt = jnp.minimum(blk + 1, NBLK - 1)
                start_in(nxt, 1 - b)
                wait_in(b)
                compute(b)

                @pl.when(blk >= 2)
                def _():
                    wait_out(b)

                start_out(blk, b)

        wait_in(0)
        wait_out(0)
        wait_out(1)

    return k(x, perm)


# speedup vs baseline: 5.2131x; 3.2723x over previous
import functools

import jax
import jax.numpy as jnp
from jax import lax
from jax.experimental import pallas as pl
from jax.experimental.pallas import tpu as pltpu
from jax.experimental.pallas import tpu_sc as plsc

L = 16


@jax.jit
def kernel(x, perm):
    N, D = x.shape
    info = plsc.get_sparse_core_info()
    NC, NS = info.num_cores, info.num_subcores
    NW = NC * NS
    assert N % NW == 0
    RW = N // NW
    R_BLK = 8
    NBLK = RW // R_BLK
    NJ = D // L
    assert NBLK % 2 == 0

    mesh = plsc.VectorSubcoreMesh(core_axis_name="c", subcore_axis_name="s")

    @functools.partial(
        pl.kernel,
        out_type=jax.ShapeDtypeStruct((N, D), jnp.float32),
        mesh=mesh,
        compiler_params=pltpu.CompilerParams(
            needs_layout_passes=False, disable_bounds_checks=True),
        scratch_types=[
            pltpu.VMEM((D,), jnp.int32),
            pltpu.VMEM((R_BLK, D), jnp.float32),
            pltpu.VMEM((R_BLK, D), jnp.float32),
            pltpu.VMEM((R_BLK, D), jnp.float32),
            pltpu.VMEM((R_BLK, D), jnp.float32),
            pltpu.SemaphoreType.DMA,
            pltpu.SemaphoreType.DMA,
            pltpu.SemaphoreType.DMA,
            pltpu.SemaphoreType.DMA,
        ],
    )
    def k(x_hbm, perm_hbm, out_hbm, perm_v, in0, in1, ou0, ou1,
          is0, is1, os0, os1):
        wid = lax.axis_index("s") * NC + lax.axis_index("c")
        base = wid * RW
        ins = (in0, in1)
        outs = (ou0, ou1)
        isems = (is0, is1)
        osems = (os0, os1)

        pltpu.sync_copy(perm_hbm, perm_v)

        def start_in(blk, b):
            pltpu.async_copy(
                x_hbm.at[pl.ds(base + blk * R_BLK, R_BLK)], ins[b], isems[b])

        def wait_in(b):
            pltpu.make_async_copy(
                x_hbm.at[pl.ds(base, R_BLK)], ins[b], isems[b]).wait()

        def start_out(blk, b):
            pltpu.async_copy(
                outs[b], out_hbm.at[pl.ds(base + blk * R_BLK, R_BLK)],
                osems[b])

        def wait_out(b):
            pltpu.make_async_copy(
                outs[b], out_hbm.at[pl.ds(base, R_BLK)], osems[b]).wait()

        def compute(b):
            ib = ins[b]
            ob = outs[b]

            @plsc.parallel_loop(0, NJ, unroll=4)
            def j_loop(j):
                col = perm_v[pl.ds(j * L, L)]
                for r in range(R_BLK):
                    rowv = jnp.full((L,), r, jnp.int32)
                    v = plsc.load_gather(ib, [rowv, col])
                    ob[r, pl.ds(j * L, L)] = v

        start_in(0, 0)

        @pl.loop(0, NBLK, step=2)
        def blk_loop(blk0):
            for b in range(2):
                blk = blk0 + b
                nxt = jnp.minimum(blk + 1, NBLK - 1)
                start_in(nxt, 1 - b)
                wait_in(b)
                compute(b)

                @pl.when(blk >= 2)
                def _():
                    wait_out(b)

                start_out(blk, b)

        wait_in(0)
        wait_out(0)
        wait_out(1)

    return k(x, perm)


# speedup vs baseline: 5.3408x; 1.0245x over previous
import functools

import jax
import jax.numpy as jnp
from jax import lax
from jax.experimental import pallas as pl
from jax.experimental.pallas import tpu as pltpu
from jax.experimental.pallas import tpu_sc as plsc

L = 16


@jax.jit
def kernel(x, perm):
    N, D = x.shape
    info = plsc.get_sparse_core_info()
    NC, NS = info.num_cores, info.num_subcores
    NW = NC * NS
    assert N % NW == 0
    RW = N // NW
    R_BLK = 4
    NBLK = RW // R_BLK
    NJ = D // L
    assert NBLK % 4 == 0

    mesh = plsc.VectorSubcoreMesh(core_axis_name="c", subcore_axis_name="s")

    @functools.partial(
        pl.kernel,
        out_type=jax.ShapeDtypeStruct((N, D), jnp.float32),
        mesh=mesh,
        compiler_params=pltpu.CompilerParams(
            needs_layout_passes=False, disable_bounds_checks=True),
        scratch_types=[
            pltpu.VMEM((D,), jnp.int32),
        ] + [pltpu.VMEM((R_BLK, D), jnp.float32)] * 8
          + [pltpu.SemaphoreType.DMA] * 8,
    )
    def k(x_hbm, perm_hbm, out_hbm, perm_v,
          in0, in1, in2, in3, ou0, ou1, ou2, ou3,
          is0, is1, is2, is3, os0, os1, os2, os3):
        wid = lax.axis_index("s") * NC + lax.axis_index("c")
        base = wid * RW
        ins = (in0, in1, in2, in3)
        outs = (ou0, ou1, ou2, ou3)
        isems = (is0, is1, is2, is3)
        osems = (os0, os1, os2, os3)

        pltpu.sync_copy(perm_hbm, perm_v)

        def start_in(blk, b):
            pltpu.async_copy(
                x_hbm.at[pl.ds(base + blk * R_BLK, R_BLK)], ins[b], isems[b])

        def wait_in(b):
            pltpu.make_async_copy(
                x_hbm.at[pl.ds(base, R_BLK)], ins[b], isems[b]).wait()

        def start_out(blk, b):
            pltpu.async_copy(
                outs[b], out_hbm.at[pl.ds(base + blk * R_BLK, R_BLK)],
                osems[b])

        def wait_out(b):
            pltpu.make_async_copy(
                outs[b], out_hbm.at[pl.ds(base, R_BLK)], osems[b]).wait()

        def compute(b):
            ib = ins[b]
            ob = outs[b]

            @plsc.parallel_loop(0, NJ, unroll=4)
            def j_loop(j):
                col = perm_v[pl.ds(j * L, L)]
                for r in range(R_BLK):
                    rowv = jnp.full((L,), r, jnp.int32)
                    v = plsc.load_gather(ib, [rowv, col])
                    ob[r, pl.ds(j * L, L)] = v

        start_in(0, 0)
        start_in(1, 1)
        start_in(2, 2)

        @pl.loop(0, NBLK, step=4)
        def blk_loop(blk0):
            for b in range(4):
                blk = blk0 + b
                nxt = jnp.minimum(blk + 3, NBLK - 1)
                start_in(nxt, (b + 3) % 4)
                wait_in(b)
                compute(b)

                @pl.when(blk >= 4)
                def _():
                    wait_out(b)

                start_out(blk, b)

        wait_in(0)
        wait_in(1)
        wait_in(2)
        for b in range(4):
            wait_out(b)

    return k(x, perm)
